# baseline (speedup 1.0000x reference)
import functools

import jax
import jax.numpy as jnp
from jax import lax
from jax.experimental import pallas as pl
from jax.experimental.pallas import tpu as pltpu
from jax.experimental.pallas import tpu_sc as plsc

_SC_PARAMS = pltpu.CompilerParams(
    use_tc_tiling_on_sc=False, needs_layout_passes=False)


def _sim_sorted_kernel(ov_ref, x_ref, w_ref, op_ref, out_ref,
                       wn_ref, *, T, CW, CPS, OVC, C):
    @pl.when(pl.program_id(0) == 0)
    def _():
        cls = lax.broadcasted_iota(jnp.int32, (CW, C), 1)

        def perm_chunk(s0):
            opj = op_ref[pl.ds(s0, CW), :]
            p = (opj == cls).astype(jnp.float32)
            ws = lax.dot_general(
                p, w_ref[...], (((1,), (0,)), ((), ())),
                precision=lax.Precision.HIGHEST,
                preferred_element_type=jnp.float32,
            )
            wn = jnp.sqrt(jnp.sum(ws * ws, axis=1, keepdims=True))
            wn_ref[pl.ds(s0, CW), :] = ws * (1.0 / jnp.maximum(wn, 1e-12))

        for j in range(T):
            perm_chunk(j * CW)

        def ov_perm(m, carry):
            perm_chunk(pl.multiple_of(CPS + m * CW, CW))
            return carry

        lax.fori_loop(0, ov_ref[OVC], ov_perm, 0)

    invs = []
    for t in range(T):
        xt = x_ref[t]
        ss = jnp.sqrt(jnp.sum(xt * xt, axis=1, keepdims=True))
        invs.append(1.0 / jnp.maximum(ss, 1e-12))

    for t in range(T):
        mt = lax.dot_general(
            x_ref[t], wn_ref[pl.ds(t * CW, CW), :], (((1,), (1,)), ((), ())),
            precision=lax.Precision.HIGHEST,
            preferred_element_type=jnp.float32,
        )
        out_ref[:, t * CW:(t + 1) * CW] = mt * invs[t]

    inv_all = jnp.concatenate(invs, axis=1)
    BN = inv_all.shape[0]
    tlane = lax.broadcasted_iota(jnp.int32, (BN, T), 1)

    def ov_body(m, carry):
        tm = ov_ref[m]
        s0 = pl.multiple_of(CPS + m * CW, CW)
        mt = lax.dot_general(
            x_ref[tm], wn_ref[pl.ds(s0, CW), :], (((1,), (1,)), ((), ())),
            precision=lax.Precision.HIGHEST,
            preferred_element_type=jnp.float32,
        )
        inv_t = jnp.sum(jnp.where(tlane == tm, inv_all, 0.0), axis=1,
                        keepdims=True)
        out_ref[:, pl.ds(s0, CW)] = mt * inv_t
        return carry

    lax.fori_loop(0, ov_ref[OVC], ov_body, 0)


def _make_select_kernel(N, CP, CPAD, K, NW):
    spw = N // NW
    nv = (K * spw) // 16
    n_groups = spw // 16
    GC = 128
    mesh = plsc.VectorSubcoreMesh(core_axis_name="c", subcore_axis_name="s")

    @functools.partial(
        pl.kernel, mesh=mesh,
        out_type=jax.ShapeDtypeStruct((N,), jnp.int32),
        compiler_params=_SC_PARAMS,
        scratch_types=[
            pltpu.VMEM((K * spw,), jnp.int32),
            pltpu.VMEM((K * spw,), jnp.int32),
            pltpu.VMEM((CPAD,), jnp.int32),
            pltpu.VMEM((K * spw,), jnp.int32),
            pltpu.VMEM((K * spw,), jnp.float32),
            pltpu.VMEM((spw,), jnp.int32),
            pltpu.SemaphoreType.DMA,
        ],
    )
    def sel(a_hbm, topk_hbm, pos_hbm, out_hbm, topkn_v, topkw_v, pos_v, idx_v,
            vals_v, lbl_v, sem):
        wid = lax.axis_index("s") * 2 + lax.axis_index("c")
        base = wid * spw
        pltpu.sync_copy(topk_hbm.at[pl.ds(base * K, spw * K)], topkn_v)
        pltpu.sync_copy(pos_hbm, pos_v)
        lane = lax.iota(jnp.int32, 16)
        vps = spw // 16

        def idx_body(v, carry):
            k = v // vps
            nl = (v % vps) * 16 + lane
            c_vec = plsc.load_gather(topkn_v, [nl * K + k])
            topkw_v[pl.ds(v * 16, 16)] = c_vec
            p_vec = plsc.load_gather(pos_v, [c_vec])
            idx_v[pl.ds(v * 16, 16)] = (base + nl) * CP + p_vec
            return carry

        lax.fori_loop(0, nv, idx_body, 0)

        copies = []
        for b in range((K * spw) // GC):
            copies.append(pltpu.make_async_copy(
                a_hbm.at[idx_v.at[pl.ds(b * GC, GC)]],
                vals_v.at[pl.ds(b * GC, GC)], sem))
        for cp in copies:
            cp.start()
        for cp in copies:
            cp.wait()

        def group_body(g, carry):
            best_val = jnp.full((16,), -jnp.inf, jnp.float32)
            best_lbl = jnp.zeros((16,), jnp.int32)
            for k in range(K):
                off = k * spw + g * 16
                c_vec = topkw_v[pl.ds(off, 16)]
                vals = vals_v[pl.ds(off, 16)]
                better = vals > best_val
                best_val = jnp.where(better, vals, best_val)
                best_lbl = jnp.where(better, c_vec, best_lbl)
            lbl_v[pl.ds(g * 16, 16)] = best_lbl
            return carry

        lax.fori_loop(0, n_groups, group_body, 0)
        pltpu.sync_copy(lbl_v, out_hbm.at[pl.ds(base, spw)])

    return sel


def _schedule(class_to_task, T, C, CW, CPS, OVC, CPAD):
    ctt = class_to_task.astype(jnp.int32)
    order = jnp.argsort(ctt)
    counts = jnp.zeros((T,), jnp.int32).at[ctt].add(1)
    group_start = jnp.cumsum(counts) - counts

    ov_cnt = jnp.maximum(counts - CW, 0)
    ovch = (ov_cnt + CW - 1) // CW
    ovcum = jnp.cumsum(ovch)
    ov_first = ovcum - ovch
    total_ov = ovcum[-1]
    m_idx = jnp.arange(OVC, dtype=jnp.int32)
    ov_task = jnp.searchsorted(ovcum, m_idx, side="right").astype(jnp.int32)
    ov_task = jnp.where(m_idx < total_ov, ov_task, -1)
    ov_task = jnp.concatenate([ov_task, total_ov[None].astype(jnp.int32)])

    rank = jnp.arange(C, dtype=jnp.int32)
    ctt_sorted = ctt[order]
    r = rank - group_start[ctt_sorted]
    slot_static = ctt_sorted * CW + r
    slot_ov = CPS + (ov_first[ctt_sorted] + (r - CW) // CW) * CW + (r - CW) % CW
    slot = jnp.where(r < CW, slot_static, slot_ov)
    pos = jnp.zeros((C,), jnp.int32).at[order].set(slot)
    pos_pad = jnp.concatenate([pos, jnp.zeros((CPAD - C,), jnp.int32)])

    CP = CPS + OVC * CW
    s = jnp.arange(CP, dtype=jnp.int32)
    is_static = s < CPS
    t_st = jnp.minimum(s // CW, T - 1)
    m_of_s = jnp.maximum(s - CPS, 0) // CW
    t_ov = ov_task[m_of_s]
    t_of_s = jnp.where(is_static, t_st, t_ov)
    t_cl = jnp.maximum(t_of_s, 0)
    rr = jnp.where(is_static, s % CW,
                   CW + (m_of_s - ov_first[t_cl]) * CW + (s - CPS) % CW)
    valid = (t_of_s >= 0) & (rr < counts[t_cl])
    src = order[jnp.clip(group_start[t_cl] + rr, 0, C - 1)]
    order_pad = jnp.where(valid, src, 0).astype(jnp.int32)
    return ov_task, pos_pad, order_pad


def kernel(X, topk_indices, class_to_task, weight, sigma):
    del sigma
    T, N, D = X.shape
    C = weight.shape[0]
    K = topk_indices.shape[1]
    BN = 512
    CW = 128
    CPS = T * CW
    OVC = 8
    CP = CPS + OVC * CW
    CPAD = 1024
    NW = 32

    ov_task, pos_pad, order_pad = _schedule(
        class_to_task, T, C, CW, CPS, OVC, CPAD)

    a = pl.pallas_call(
        functools.partial(_sim_sorted_kernel, T=T, CW=CW, CPS=CPS, OVC=OVC,
                          C=C),
        grid=(N // BN,),
        in_specs=[
            pl.BlockSpec(memory_space=pltpu.SMEM),
            pl.BlockSpec((T, BN, D), lambda i: (0, i, 0)),
            pl.BlockSpec((C, D), lambda i: (0, 0)),
            pl.BlockSpec((CP, 1), lambda i: (0, 0)),
        ],
        out_specs=pl.BlockSpec((BN, CP), lambda i: (i, 0)),
        out_shape=jax.ShapeDtypeStruct((N, CP), jnp.float32),
        scratch_shapes=[
            pltpu.VMEM((CP, D), jnp.float32),
        ],
        compiler_params=pltpu.CompilerParams(
            vmem_limit_bytes=112 * 1024 * 1024),
    )(ov_task, X, weight, order_pad.reshape(CP, 1))

    return _make_select_kernel(N, CP, CPAD, K, NW)(
        a.reshape(-1), topk_indices.astype(jnp.int32).reshape(-1), pos_pad)


# speedup vs baseline: 1.0772x; 1.0772x over previous
import functools

import jax
import jax.numpy as jnp
from jax import lax
from jax.experimental import pallas as pl
from jax.experimental.pallas import tpu as pltpu
from jax.experimental.pallas import tpu_sc as plsc

_SC_PARAMS = pltpu.CompilerParams(
    use_tc_tiling_on_sc=False, needs_layout_passes=False)


def _perm_kernel(ov_ref, w_ref, op_ref, wn_ref, *, T, CW, CPS, OVC, C):
    cls = lax.broadcasted_iota(jnp.int32, (CW, C), 1)

    def perm_chunk(s0):
        opj = op_ref[pl.ds(s0, CW), :]
        p = (opj == cls).astype(jnp.float32)
        ws = lax.dot_general(
            p, w_ref[...], (((1,), (0,)), ((), ())),
            precision=lax.Precision.HIGHEST,
            preferred_element_type=jnp.float32,
        )
        wn = jnp.sqrt(jnp.sum(ws * ws, axis=1, keepdims=True))
        wn_ref[pl.ds(s0, CW), :] = ws * (1.0 / jnp.maximum(wn, 1e-12))

    for j in range(T):
        perm_chunk(j * CW)

    def ov_perm(m, carry):
        perm_chunk(pl.multiple_of(CPS + m * CW, CW))
        return carry

    lax.fori_loop(0, ov_ref[OVC], ov_perm, 0)


def _sim_sorted_kernel(ov_ref, x_ref, wn_ref, out_ref, *, T, CW, CPS, OVC):
    invs = []
    for t in range(T):
        xt = x_ref[t]
        ss = jnp.sqrt(jnp.sum(xt * xt, axis=1, keepdims=True))
        invs.append(1.0 / jnp.maximum(ss, 1e-12))

    for t in range(T):
        mt = lax.dot_general(
            x_ref[t], wn_ref[pl.ds(t * CW, CW), :], (((1,), (1,)), ((), ())),
            precision=lax.Precision.HIGHEST,
            preferred_element_type=jnp.float32,
        )
        out_ref[:, t * CW:(t + 1) * CW] = mt * invs[t]

    inv_all = jnp.concatenate(invs, axis=1)
    BN = inv_all.shape[0]
    tlane = lax.broadcasted_iota(jnp.int32, (BN, T), 1)

    def ov_body(m, carry):
        tm = ov_ref[m]
        s0 = pl.multiple_of(CPS + m * CW, CW)
        mt = lax.dot_general(
            x_ref[tm], wn_ref[pl.ds(s0, CW), :], (((1,), (1,)), ((), ())),
            precision=lax.Precision.HIGHEST,
            preferred_element_type=jnp.float32,
        )
        inv_t = jnp.sum(jnp.where(tlane == tm, inv_all, 0.0), axis=1,
                        keepdims=True)
        out_ref[:, pl.ds(s0, CW)] = mt * inv_t
        return carry

    lax.fori_loop(0, ov_ref[OVC], ov_body, 0)


def _make_select_kernel(N, CP, CPAD, K, NW):
    spw = N // NW
    nv = (K * spw) // 16
    n_groups = spw // 16
    GC = 128
    mesh = plsc.VectorSubcoreMesh(core_axis_name="c", subcore_axis_name="s")

    @functools.partial(
        pl.kernel, mesh=mesh,
        out_type=jax.ShapeDtypeStruct((N,), jnp.int32),
        compiler_params=_SC_PARAMS,
        scratch_types=[
            pltpu.VMEM((K * spw,), jnp.int32),
            pltpu.VMEM((K * spw,), jnp.int32),
            pltpu.VMEM((CPAD,), jnp.int32),
            pltpu.VMEM((K * spw,), jnp.int32),
            pltpu.VMEM((K * spw,), jnp.float32),
            pltpu.VMEM((spw,), jnp.int32),
            pltpu.SemaphoreType.DMA,
        ],
    )
    def sel(a_hbm, topk_hbm, pos_hbm, out_hbm, topkn_v, topkw_v, pos_v, idx_v,
            vals_v, lbl_v, sem):
        wid = lax.axis_index("s") * 2 + lax.axis_index("c")
        base = wid * spw
        pltpu.sync_copy(topk_hbm.at[pl.ds(base * K, spw * K)], topkn_v)
        pltpu.sync_copy(pos_hbm, pos_v)
        lane = lax.iota(jnp.int32, 16)
        vps = spw // 16

        def idx_body(v, carry):
            k = v // vps
            nl = (v % vps) * 16 + lane
            c_vec = plsc.load_gather(topkn_v, [nl * K + k])
            topkw_v[pl.ds(v * 16, 16)] = c_vec
            p_vec = plsc.load_gather(pos_v, [c_vec])
            idx_v[pl.ds(v * 16, 16)] = (base + nl) * CP + p_vec
            return carry

        lax.fori_loop(0, nv, idx_body, 0)

        copies = []
        for b in range((K * spw) // GC):
            copies.append(pltpu.make_async_copy(
                a_hbm.at[idx_v.at[pl.ds(b * GC, GC)]],
                vals_v.at[pl.ds(b * GC, GC)], sem))
        for cp in copies:
            cp.start()
        for cp in copies:
            cp.wait()

        def group_body(g, carry):
            best_val = jnp.full((16,), -jnp.inf, jnp.float32)
            best_lbl = jnp.zeros((16,), jnp.int32)
            for k in range(K):
                off = k * spw + g * 16
                c_vec = topkw_v[pl.ds(off, 16)]
                vals = vals_v[pl.ds(off, 16)]
                better = vals > best_val
                best_val = jnp.where(better, vals, best_val)
                best_lbl = jnp.where(better, c_vec, best_lbl)
            lbl_v[pl.ds(g * 16, 16)] = best_lbl
            return carry

        lax.fori_loop(0, n_groups, group_body, 0)
        pltpu.sync_copy(lbl_v, out_hbm.at[pl.ds(base, spw)])

    return sel


def _schedule(class_to_task, T, C, CW, CPS, OVC, CPAD):
    ctt = class_to_task.astype(jnp.int32)
    order = jnp.argsort(ctt)
    counts = jnp.zeros((T,), jnp.int32).at[ctt].add(1)
    group_start = jnp.cumsum(counts) - counts

    ov_cnt = jnp.maximum(counts - CW, 0)
    ovch = (ov_cnt + CW - 1) // CW
    ovcum = jnp.cumsum(ovch)
    ov_first = ovcum - ovch
    total_ov = ovcum[-1]
    m_idx = jnp.arange(OVC, dtype=jnp.int32)
    ov_task = jnp.searchsorted(ovcum, m_idx, side="right").astype(jnp.int32)
    ov_task = jnp.where(m_idx < total_ov, ov_task, -1)
    ov_task = jnp.concatenate([ov_task, total_ov[None].astype(jnp.int32)])

    rank = jnp.arange(C, dtype=jnp.int32)
    ctt_sorted = ctt[order]
    r = rank - group_start[ctt_sorted]
    slot_static = ctt_sorted * CW + r
    slot_ov = CPS + (ov_first[ctt_sorted] + (r - CW) // CW) * CW + (r - CW) % CW
    slot = jnp.where(r < CW, slot_static, slot_ov)
    pos = jnp.zeros((C,), jnp.int32).at[order].set(slot)
    pos_pad = jnp.concatenate([pos, jnp.zeros((CPAD - C,), jnp.int32)])

    CP = CPS + OVC * CW
    s = jnp.arange(CP, dtype=jnp.int32)
    is_static = s < CPS
    t_st = jnp.minimum(s // CW, T - 1)
    m_of_s = jnp.maximum(s - CPS, 0) // CW
    t_ov = ov_task[m_of_s]
    t_of_s = jnp.where(is_static, t_st, t_ov)
    t_cl = jnp.maximum(t_of_s, 0)
    rr = jnp.where(is_static, s % CW,
                   CW + (m_of_s - ov_first[t_cl]) * CW + (s - CPS) % CW)
    valid = (t_of_s >= 0) & (rr < counts[t_cl])
    src = order[jnp.clip(group_start[t_cl] + rr, 0, C - 1)]
    order_pad = jnp.where(valid, src, 0).astype(jnp.int32)
    return ov_task, pos_pad, order_pad


def kernel(X, topk_indices, class_to_task, weight, sigma):
    del sigma
    T, N, D = X.shape
    C = weight.shape[0]
    K = topk_indices.shape[1]
    BN = 256
    CW = 128
    CPS = T * CW
    OVC = 8
    CP = CPS + OVC * CW
    CPAD = 1024
    NW = 32

    ov_task, pos_pad, order_pad = _schedule(
        class_to_task, T, C, CW, CPS, OVC, CPAD)

    wn = pl.pallas_call(
        functools.partial(_perm_kernel, T=T, CW=CW, CPS=CPS, OVC=OVC, C=C),
        grid=(1,),
        in_specs=[
            pl.BlockSpec(memory_space=pltpu.SMEM),
            pl.BlockSpec((C, D), lambda i: (0, 0)),
            pl.BlockSpec((CP, 1), lambda i: (0, 0)),
        ],
        out_specs=pl.BlockSpec((CP, D), lambda i: (0, 0)),
        out_shape=jax.ShapeDtypeStruct((CP, D), jnp.float32),
    )(ov_task, weight, order_pad.reshape(CP, 1))

    a = pl.pallas_call(
        functools.partial(_sim_sorted_kernel, T=T, CW=CW, CPS=CPS, OVC=OVC),
        grid=(N // BN,),
        in_specs=[
            pl.BlockSpec(memory_space=pltpu.SMEM),
            pl.BlockSpec((T, BN, D), lambda i: (0, i, 0)),
            pl.BlockSpec((CP, D), lambda i: (0, 0)),
        ],
        out_specs=pl.BlockSpec((BN, CP), lambda i: (i, 0)),
        out_shape=jax.ShapeDtypeStruct((N, CP), jnp.float32),
    )(ov_task, X, wn)

    return _make_select_kernel(N, CP, CPAD, K, NW)(
        a.reshape(-1), topk_indices.astype(jnp.int32).reshape(-1), pos_pad)


# speedup vs baseline: 1.3165x; 1.2221x over previous
import functools

import jax
import jax.numpy as jnp
from jax import lax
from jax.experimental import pallas as pl
from jax.experimental.pallas import tpu as pltpu
from jax.experimental.pallas import tpu_sc as plsc

_SC_PARAMS = pltpu.CompilerParams(
    use_tc_tiling_on_sc=False, needs_layout_passes=False)


def _perm_kernel(ov_ref, w_ref, op_ref, wn_ref, *, T, CW, CPS, OVC, C):
    cls = lax.broadcasted_iota(jnp.int32, (CW, C), 1)

    def perm_chunk(s0):
        opj = op_ref[pl.ds(s0, CW), :]
        p = (opj == cls).astype(jnp.float32)
        ws = lax.dot_general(
            p, w_ref[...], (((1,), (0,)), ((), ())),
            precision=lax.Precision.HIGHEST,
            preferred_element_type=jnp.float32,
        )
        wn = jnp.sqrt(jnp.sum(ws * ws, axis=1, keepdims=True))
        wn_ref[pl.ds(s0, CW), :] = ws * (1.0 / jnp.maximum(wn, 1e-12))

    for j in range(T):
        perm_chunk(j * CW)

    def ov_perm(m, carry):
        perm_chunk(pl.multiple_of(CPS + m * CW, CW))
        return carry

    lax.fori_loop(0, ov_ref[OVC], ov_perm, 0)


def _sim_sorted_kernel(ov_ref, x_ref, wn_ref, out_ref, *, T, CW, CPS, OVC):
    invs = []
    for t in range(T):
        xt = x_ref[t]
        ss = jnp.sqrt(jnp.sum(xt * xt, axis=1, keepdims=True))
        invs.append(1.0 / jnp.maximum(ss, 1e-12))

    for t in range(T):
        mt = lax.dot_general(
            x_ref[t], wn_ref[pl.ds(t * CW, CW), :], (((1,), (1,)), ((), ())),
            precision=lax.Precision.HIGHEST,
            preferred_element_type=jnp.float32,
        )
        out_ref[:, t * CW:(t + 1) * CW] = mt * invs[t]

    inv_all = jnp.concatenate(invs, axis=1)
    BN = inv_all.shape[0]
    tlane = lax.broadcasted_iota(jnp.int32, (BN, T), 1)

    def ov_body(m, carry):
        tm = ov_ref[m]
        s0 = pl.multiple_of(CPS + m * CW, CW)
        mt = lax.dot_general(
            x_ref[tm], wn_ref[pl.ds(s0, CW), :], (((1,), (1,)), ((), ())),
            precision=lax.Precision.HIGHEST,
            preferred_element_type=jnp.float32,
        )
        inv_t = jnp.sum(jnp.where(tlane == tm, inv_all, 0.0), axis=1,
                        keepdims=True)
        out_ref[:, pl.ds(s0, CW)] = mt * inv_t
        return carry

    lax.fori_loop(0, ov_ref[OVC], ov_body, 0)


def _make_select_kernel(N, CP, CPAD, K, NW):
    spw = N // NW
    nv = (K * spw) // 16
    n_groups = spw // 16
    GC = 128
    mesh = plsc.VectorSubcoreMesh(core_axis_name="c", subcore_axis_name="s")

    @functools.partial(
        pl.kernel, mesh=mesh,
        out_type=jax.ShapeDtypeStruct((N,), jnp.int32),
        compiler_params=_SC_PARAMS,
        scratch_types=[
            pltpu.VMEM((K * spw,), jnp.int32),
            pltpu.VMEM((K * spw,), jnp.int32),
            pltpu.VMEM((CPAD,), jnp.int32),
            pltpu.VMEM((K * spw,), jnp.int32),
            pltpu.VMEM((K * spw,), jnp.float32),
            pltpu.VMEM((spw,), jnp.int32),
            pltpu.SemaphoreType.DMA,
        ],
    )
    def sel(a_hbm, topk_hbm, pos_hbm, out_hbm, topkn_v, topkw_v, pos_v, idx_v,
            vals_v, lbl_v, sem):
        wid = lax.axis_index("s") * 2 + lax.axis_index("c")
        base = wid * spw
        pltpu.sync_copy(topk_hbm.at[pl.ds(base * K, spw * K)], topkn_v)
        pltpu.sync_copy(pos_hbm, pos_v)
        lane = lax.iota(jnp.int32, 16)
        vps = spw // 16

        def idx_body(v, carry):
            k = v // vps
            nl = (v % vps) * 16 + lane
            c_vec = plsc.load_gather(topkn_v, [nl * K + k])
            topkw_v[pl.ds(v * 16, 16)] = c_vec
            p_vec = plsc.load_gather(pos_v, [c_vec])
            idx_v[pl.ds(v * 16, 16)] = (base + nl) * CP + p_vec
            return carry

        lax.fori_loop(0, nv, idx_body, 0)

        copies = []
        for b in range((K * spw) // GC):
            copies.append(pltpu.make_async_copy(
                a_hbm.at[idx_v.at[pl.ds(b * GC, GC)]],
                vals_v.at[pl.ds(b * GC, GC)], sem))
        for cp in copies:
            cp.start()
        for cp in copies:
            cp.wait()

        def group_body(g, carry):
            best_val = jnp.full((16,), -jnp.inf, jnp.float32)
            best_lbl = jnp.zeros((16,), jnp.int32)
            for k in range(K):
                off = k * spw + g * 16
                c_vec = topkw_v[pl.ds(off, 16)]
                vals = vals_v[pl.ds(off, 16)]
                better = vals > best_val
                best_val = jnp.where(better, vals, best_val)
                best_lbl = jnp.where(better, c_vec, best_lbl)
            lbl_v[pl.ds(g * 16, 16)] = best_lbl
            return carry

        lax.fori_loop(0, n_groups, group_body, 0)
        pltpu.sync_copy(lbl_v, out_hbm.at[pl.ds(base, spw)])

    return sel


def _schedule(class_to_task, T, C, CW, CPS, OVC, CPAD):
    ctt = class_to_task.astype(jnp.int32)
    onehot = (ctt[:, None] == jnp.arange(T, dtype=jnp.int32)[None, :])
    cum = jnp.cumsum(onehot.astype(jnp.int32), axis=0)
    counts = cum[-1]
    r = jnp.take_along_axis(cum, ctt[:, None], axis=1)[:, 0] - 1

    ov_cnt = jnp.maximum(counts - CW, 0)
    ovch = (ov_cnt + CW - 1) // CW
    ovcum = jnp.cumsum(ovch)
    ov_first = ovcum - ovch
    total_ov = ovcum[-1]
    m_idx = jnp.arange(OVC, dtype=jnp.int32)
    ov_task = jnp.searchsorted(ovcum, m_idx, side="right").astype(jnp.int32)
    ov_task = jnp.where(m_idx < total_ov, ov_task, -1)
    ov_task = jnp.concatenate([ov_task, total_ov[None].astype(jnp.int32)])

    slot_static = ctt * CW + r
    slot_ov = CPS + (ov_first[ctt] + (r - CW) // CW) * CW + (r - CW) % CW
    pos = jnp.where(r < CW, slot_static, slot_ov).astype(jnp.int32)
    pos_pad = jnp.concatenate([pos, jnp.zeros((CPAD - C,), jnp.int32)])

    CP = CPS + OVC * CW
    order_pad = (jnp.zeros((CP,), jnp.int32)
                 .at[pos].set(jnp.arange(C, dtype=jnp.int32)))
    return ov_task, pos_pad, order_pad


def kernel(X, topk_indices, class_to_task, weight, sigma):
    del sigma
    T, N, D = X.shape
    C = weight.shape[0]
    K = topk_indices.shape[1]
    BN = 256
    CW = 128
    CPS = T * CW
    OVC = 8
    CP = CPS + OVC * CW
    CPAD = 1024
    NW = 32

    ov_task, pos_pad, order_pad = _schedule(
        class_to_task, T, C, CW, CPS, OVC, CPAD)

    wn = pl.pallas_call(
        functools.partial(_perm_kernel, T=T, CW=CW, CPS=CPS, OVC=OVC, C=C),
        grid=(1,),
        in_specs=[
            pl.BlockSpec(memory_space=pltpu.SMEM),
            pl.BlockSpec((C, D), lambda i: (0, 0)),
            pl.BlockSpec((CP, 1), lambda i: (0, 0)),
        ],
        out_specs=pl.BlockSpec((CP, D), lambda i: (0, 0)),
        out_shape=jax.ShapeDtypeStruct((CP, D), jnp.float32),
    )(ov_task, weight, order_pad.reshape(CP, 1))

    a = pl.pallas_call(
        functools.partial(_sim_sorted_kernel, T=T, CW=CW, CPS=CPS, OVC=OVC),
        grid=(N // BN,),
        in_specs=[
            pl.BlockSpec(memory_space=pltpu.SMEM),
            pl.BlockSpec((T, BN, D), lambda i: (0, i, 0)),
            pl.BlockSpec((CP, D), lambda i: (0, 0)),
        ],
        out_specs=pl.BlockSpec((BN, CP), lambda i: (i, 0)),
        out_shape=jax.ShapeDtypeStruct((N, CP), jnp.float32),
    )(ov_task, X, wn)

    return _make_select_kernel(N, CP, CPAD, K, NW)(
        a.reshape(-1), topk_indices.astype(jnp.int32).reshape(-1), pos_pad)
